# Initial kernel scaffold; baseline (speedup 1.0000x reference)
import jax
import jax.numpy as jnp
from jax.experimental import pallas as pl


def kernel(x, input_patterns, process_input_weights, process_values, output_input_weights, output_patterns):
    raise NotImplementedError("write your pallas kernel here")


# speedup vs baseline: 4.8388x; 4.8388x over previous
import functools

import jax
import jax.numpy as jnp
from jax.experimental import pallas as pl
from jax.experimental.pallas import tpu as pltpu

_B, _S, _D_MODEL = 4, 2048, 1024
_N_IN, _N_PROC, _N_OUT, _D_PV = 4096, 2048, 4096, 512
_K_IN, _K_PROC, _K_OUT = _N_IN // 8, _N_PROC // 8, _N_OUT // 8


def _gelu(v):
    return 0.5 * v * (1.0 + jax.lax.erf(v * 0.7071067811865476))


def _kth_largest(acts, k):
    bits = jax.lax.bitcast_convert_type(acts, jnp.uint32)
    top = jnp.uint32(0x80000000)
    codes = jnp.where(bits >= top, ~bits, bits | top)

    def body(i, res):
        cand = res | (jnp.uint32(1) << (jnp.uint32(31) - i.astype(jnp.uint32)))
        cnt = jnp.sum((codes >= cand).astype(jnp.int32), axis=1, keepdims=True)
        return jnp.where(cnt >= k, cand, res)

    res = jax.lax.fori_loop(0, 32, body, jnp.zeros((acts.shape[0], 1), jnp.uint32))
    thr_bits = jnp.where(res >= top, res ^ top, ~res)
    return jax.lax.bitcast_convert_type(thr_bits, jnp.float32)


def _stage1_body(x_ref, w_ref, out_ref):
    s = pl.program_id(2)
    scores = jax.lax.dot_general(
        x_ref[...], w_ref[...], (((1,), (1,)), ((), ())),
        preferred_element_type=jnp.float32)
    partial = jnp.sum(_gelu(scores), axis=0, keepdims=True)[None]

    @pl.when(s == 0)
    def _():
        out_ref[...] = jnp.zeros_like(out_ref)

    out_ref[...] += partial

    @pl.when(s == pl.num_programs(2) - 1)
    def _():
        out_ref[...] = out_ref[...] * (1.0 / _S)


def _stage1(x, input_patterns):
    TS, TN = 512, 512
    return pl.pallas_call(
        _stage1_body,
        grid=(_B, _N_IN // TN, _S // TS),
        in_specs=[
            pl.BlockSpec((None, TS, _D_MODEL), lambda b, n, s: (b, s, 0)),
            pl.BlockSpec((TN, _D_MODEL), lambda b, n, s: (n, 0)),
        ],
        out_specs=pl.BlockSpec((1, 1, TN), lambda b, n, s: (b, 0, n)),
        out_shape=jax.ShapeDtypeStruct((_B, 1, _N_IN), jnp.float32),
        compiler_params=pltpu.CompilerParams(
            dimension_semantics=("parallel", "parallel", "arbitrary")),
    )(x, input_patterns).reshape(_B, _N_IN)


def _mask_body(k, acts_ref, out_ref):
    acts = acts_ref[...]
    thr = _kth_largest(acts, k)
    out_ref[...] = jnp.where(acts >= thr, acts, 0.0)


def _masked_repr(acts, k):
    return pl.pallas_call(
        functools.partial(_mask_body, k),
        out_shape=jax.ShapeDtypeStruct(acts.shape, jnp.float32),
    )(acts)


def _stage2_body(repr_ref, w_ref, out_ref):
    scores = jax.lax.dot_general(
        repr_ref[...], w_ref[...], (((1,), (1,)), ((), ())),
        preferred_element_type=jnp.float32)
    out_ref[...] = _gelu(scores)


def _stage2(input_repr, process_input_weights):
    TP = 512
    return pl.pallas_call(
        _stage2_body,
        grid=(_N_PROC // TP,),
        in_specs=[
            pl.BlockSpec((_B, _N_IN), lambda p: (0, 0)),
            pl.BlockSpec((TP, _N_IN), lambda p: (p, 0)),
        ],
        out_specs=pl.BlockSpec((_B, TP), lambda p: (0, p)),
        out_shape=jax.ShapeDtypeStruct((_B, _N_PROC), jnp.float32),
    )(input_repr, process_input_weights)


def _stage2b_body(pacts_ref, pv_ref, out_ref):
    pacts = pacts_ref[...]
    thr = _kth_largest(pacts, _K_PROC)
    mask = pacts >= thr
    rowmax = jnp.max(pacts, axis=1, keepdims=True)
    e = jnp.where(mask, jnp.exp(pacts - rowmax), 0.0)
    w = e / jnp.sum(e, axis=1, keepdims=True)
    out_ref[...] = jax.lax.dot_general(
        w, pv_ref[...], (((1,), (0,)), ((), ())),
        preferred_element_type=jnp.float32)


def _stage2b(pacts, process_values):
    return pl.pallas_call(
        _stage2b_body,
        out_shape=jax.ShapeDtypeStruct((_B, _D_PV), jnp.float32),
    )(pacts, process_values)


def _stage3a_body(agg_ref, w_ref, out_ref):
    scores = jax.lax.dot_general(
        agg_ref[...], w_ref[...], (((1,), (1,)), ((), ())),
        preferred_element_type=jnp.float32)
    oacts = _gelu(scores)
    thr = _kth_largest(oacts, _K_OUT)
    out_ref[...] = jnp.where(oacts >= thr, oacts, 0.0)


def _stage3a(agg, output_input_weights):
    return pl.pallas_call(
        _stage3a_body,
        out_shape=jax.ShapeDtypeStruct((_B, _N_OUT), jnp.float32),
    )(agg, output_input_weights)


def _stage3b_body(macts_ref, p_ref, out_ref):
    out_ref[...] = jax.lax.dot_general(
        macts_ref[...], p_ref[...], (((1,), (0,)), ((), ())),
        preferred_element_type=jnp.float32)


def _stage3b(macts, output_patterns):
    TD = 512
    return pl.pallas_call(
        _stage3b_body,
        grid=(_D_MODEL // TD,),
        in_specs=[
            pl.BlockSpec((_B, _N_OUT), lambda d: (0, 0)),
            pl.BlockSpec((_N_OUT, TD), lambda d: (0, d)),
        ],
        out_specs=pl.BlockSpec((_B, TD), lambda d: (0, d)),
        out_shape=jax.ShapeDtypeStruct((_B, _D_MODEL), jnp.float32),
    )(macts, output_patterns)


def kernel(x, input_patterns, process_input_weights, process_values,
           output_input_weights, output_patterns):
    acts_seq = _stage1(x, input_patterns)
    input_repr = _masked_repr(acts_seq, _K_IN)
    pacts = _stage2(input_repr, process_input_weights)
    agg = _stage2b(pacts, process_values)
    macts = _stage3a(agg, output_input_weights)
    out_row = _stage3b(macts, output_patterns)
    return jnp.broadcast_to(out_row[:, None, :], (_B, _S, _D_MODEL))


# speedup vs baseline: 7.6440x; 1.5797x over previous
import functools

import jax
import jax.numpy as jnp
from jax.experimental import pallas as pl
from jax.experimental.pallas import tpu as pltpu

_B, _S, _D_MODEL = 4, 2048, 1024
_N_IN, _N_PROC, _N_OUT, _D_PV = 4096, 2048, 4096, 512
_K_IN, _K_PROC, _K_OUT = _N_IN // 8, _N_PROC // 8, _N_OUT // 8


def _gelu(v):
    return 0.5 * v * (1.0 + jax.lax.erf(v * 0.7071067811865476))


def _kth_largest(acts, k):
    bits = jax.lax.bitcast_convert_type(acts, jnp.uint32)
    top = jnp.uint32(0x80000000)
    codes = jnp.where(bits >= top, ~bits, bits | top)

    def body(i, res):
        cand = res | (jnp.uint32(1) << (jnp.uint32(31) - i.astype(jnp.uint32)))
        cnt = jnp.sum((codes >= cand).astype(jnp.int32), axis=1, keepdims=True)
        return jnp.where(cnt >= k, cand, res)

    res = jax.lax.fori_loop(0, 32, body, jnp.zeros((acts.shape[0], 1), jnp.uint32))
    thr_bits = jnp.where(res >= top, res ^ top, ~res)
    return jax.lax.bitcast_convert_type(thr_bits, jnp.float32)


def _stage1_body(x_ref, w_ref, out_ref):
    s = pl.program_id(1)
    scores = jax.lax.dot_general(
        x_ref[...], w_ref[...], (((1,), (1,)), ((), ())),
        preferred_element_type=jnp.float32)
    partial = jnp.sum(_gelu(scores), axis=0, keepdims=True)[None]

    @pl.when(s == 0)
    def _():
        out_ref[...] = jnp.zeros_like(out_ref)

    out_ref[...] += partial

    @pl.when(s == pl.num_programs(1) - 1)
    def _():
        out_ref[...] = out_ref[...] * (1.0 / _S)


def _stage1(x, input_patterns):
    TS = 512
    return pl.pallas_call(
        _stage1_body,
        grid=(_B, _S // TS),
        in_specs=[
            pl.BlockSpec((None, TS, _D_MODEL), lambda b, s: (b, s, 0)),
            pl.BlockSpec((_N_IN, _D_MODEL), lambda b, s: (0, 0)),
        ],
        out_specs=pl.BlockSpec((1, 1, _N_IN), lambda b, s: (b, 0, 0)),
        out_shape=jax.ShapeDtypeStruct((_B, 1, _N_IN), jnp.float32),
        compiler_params=pltpu.CompilerParams(
            dimension_semantics=("parallel", "arbitrary")),
    )(x, input_patterns).reshape(_B, _N_IN)


def _mask_body(k, acts_ref, out_ref):
    acts = acts_ref[...]
    thr = _kth_largest(acts, k)
    out_ref[...] = jnp.where(acts >= thr, acts, 0.0)


def _masked_repr(acts, k):
    return pl.pallas_call(
        functools.partial(_mask_body, k),
        out_shape=jax.ShapeDtypeStruct(acts.shape, jnp.float32),
    )(acts)


def _stage2_body(repr_ref, w_ref, out_ref):
    scores = jax.lax.dot_general(
        repr_ref[...], w_ref[...], (((1,), (1,)), ((), ())),
        preferred_element_type=jnp.float32)
    out_ref[...] = _gelu(scores)


def _stage2(input_repr, process_input_weights):
    TP = 512
    return pl.pallas_call(
        _stage2_body,
        grid=(_N_PROC // TP,),
        in_specs=[
            pl.BlockSpec((_B, _N_IN), lambda p: (0, 0)),
            pl.BlockSpec((TP, _N_IN), lambda p: (p, 0)),
        ],
        out_specs=pl.BlockSpec((_B, TP), lambda p: (0, p)),
        out_shape=jax.ShapeDtypeStruct((_B, _N_PROC), jnp.float32),
    )(input_repr, process_input_weights)


def _stage2b_body(pacts_ref, pv_ref, out_ref):
    pacts = pacts_ref[...]
    thr = _kth_largest(pacts, _K_PROC)
    mask = pacts >= thr
    rowmax = jnp.max(pacts, axis=1, keepdims=True)
    e = jnp.where(mask, jnp.exp(pacts - rowmax), 0.0)
    w = e / jnp.sum(e, axis=1, keepdims=True)
    out_ref[...] = jax.lax.dot_general(
        w, pv_ref[...], (((1,), (0,)), ((), ())),
        preferred_element_type=jnp.float32)


def _stage2b(pacts, process_values):
    return pl.pallas_call(
        _stage2b_body,
        out_shape=jax.ShapeDtypeStruct((_B, _D_PV), jnp.float32),
    )(pacts, process_values)


def _stage3a_body(agg_ref, w_ref, out_ref):
    scores = jax.lax.dot_general(
        agg_ref[...], w_ref[...], (((1,), (1,)), ((), ())),
        preferred_element_type=jnp.float32)
    oacts = _gelu(scores)
    thr = _kth_largest(oacts, _K_OUT)
    out_ref[...] = jnp.where(oacts >= thr, oacts, 0.0)


def _stage3a(agg, output_input_weights):
    return pl.pallas_call(
        _stage3a_body,
        out_shape=jax.ShapeDtypeStruct((_B, _N_OUT), jnp.float32),
    )(agg, output_input_weights)


def _stage3b_body(macts_ref, p_ref, out_ref):
    out_ref[...] = jax.lax.dot_general(
        macts_ref[...], p_ref[...], (((1,), (0,)), ((), ())),
        preferred_element_type=jnp.float32)


def _stage3b(macts, output_patterns):
    TD = 512
    return pl.pallas_call(
        _stage3b_body,
        grid=(_D_MODEL // TD,),
        in_specs=[
            pl.BlockSpec((_B, _N_OUT), lambda d: (0, 0)),
            pl.BlockSpec((_N_OUT, TD), lambda d: (0, d)),
        ],
        out_specs=pl.BlockSpec((_B, TD), lambda d: (0, d)),
        out_shape=jax.ShapeDtypeStruct((_B, _D_MODEL), jnp.float32),
    )(macts, output_patterns)


def kernel(x, input_patterns, process_input_weights, process_values,
           output_input_weights, output_patterns):
    acts_seq = _stage1(x, input_patterns)
    input_repr = _masked_repr(acts_seq, _K_IN)
    pacts = _stage2(input_repr, process_input_weights)
    agg = _stage2b(pacts, process_values)
    macts = _stage3a(agg, output_input_weights)
    out_row = _stage3b(macts, output_patterns)
    return jnp.broadcast_to(out_row[:, None, :], (_B, _S, _D_MODEL))
